# Initial kernel scaffold; baseline (speedup 1.0000x reference)
import jax
import jax.numpy as jnp
from jax.experimental import pallas as pl


def kernel(x, token_embedding, pos_embedding):
    raise NotImplementedError("write your pallas kernel here")


# speedup vs baseline: 1.2418x; 1.2418x over previous
import functools

import jax
import jax.numpy as jnp
from jax import lax
from jax.experimental import pallas as pl
from jax.experimental.pallas import tpu as pltpu
from jax.experimental.pallas import tpu_sc as plsc

_NUM_CORES = 2
_NUM_SUBCORES = 16
_NW = _NUM_CORES * _NUM_SUBCORES
_HALF = 100
_LANES = 16


def kernel(x, token_embedding, pos_embedding):
    B, S = x.shape
    V, D = token_embedding.shape
    per_w = B // _NW
    x3 = x.reshape(B, S // _HALF, _HALF).astype(jnp.int32)

    mesh = plsc.VectorSubcoreMesh(core_axis_name="c", subcore_axis_name="s")

    @functools.partial(
        pl.kernel,
        out_type=jax.ShapeDtypeStruct((B, S, D), jnp.float32),
        mesh=mesh,
        compiler_params=pltpu.CompilerParams(use_tc_tiling_on_sc=False),
        scratch_types=[
            pltpu.VMEM((S // _HALF, _HALF), jnp.int32),
            pltpu.VMEM((S, D), jnp.float32),
            pltpu.VMEM((S, D), jnp.float32),
            pltpu.SemaphoreType.DMA,
        ],
    )
    def emb_kernel(x_hbm, tok_hbm, pos_hbm, out_hbm, idx_v, rows_v, pos_v, sem):
        wid = lax.axis_index("s") * _NUM_CORES + lax.axis_index("c")
        base = wid * per_w
        pltpu.sync_copy(pos_hbm.at[pl.ds(0, S)], pos_v)

        @pl.loop(0, per_w)
        def _(j):
            seq = base + j
            pltpu.sync_copy(x_hbm.at[seq], idx_v)
            pltpu.async_copy(
                tok_hbm.at[idx_v.at[0]], rows_v.at[pl.ds(0, _HALF)], sem
            ).wait()
            pltpu.async_copy(
                tok_hbm.at[idx_v.at[1]], rows_v.at[pl.ds(_HALF, _HALF)], sem
            ).wait()

            @pl.loop(0, S)
            def _(r):
                for c in range(0, D, _LANES):
                    slc = (pl.ds(r, 1), pl.ds(c, _LANES))
                    rows_v.at[*slc][...] = rows_v.at[*slc][...] + pos_v.at[*slc][...]

            pltpu.sync_copy(rows_v, out_hbm.at[seq])

    return emb_kernel(x3, token_embedding, pos_embedding)


# speedup vs baseline: 1.3684x; 1.1020x over previous
import functools

import jax
import jax.numpy as jnp
from jax import lax
from jax.experimental import pallas as pl
from jax.experimental.pallas import tpu as pltpu
from jax.experimental.pallas import tpu_sc as plsc

_NUM_CORES = 2
_NUM_SUBCORES = 16
_NW = _NUM_CORES * _NUM_SUBCORES
_HALF = 100
_LANES = 16
_NBUF = 4


def kernel(x, token_embedding, pos_embedding):
    B, S = x.shape
    V, D = token_embedding.shape
    n_seq = B // _NW
    x4 = x.reshape(_NW, n_seq * (S // _HALF), _HALF).astype(jnp.int32)

    mesh = plsc.VectorSubcoreMesh(core_axis_name="c", subcore_axis_name="s")

    @functools.partial(
        pl.kernel,
        out_type=jax.ShapeDtypeStruct((B, S, D), jnp.float32),
        mesh=mesh,
        compiler_params=pltpu.CompilerParams(use_tc_tiling_on_sc=False),
        scratch_types=[
            pltpu.VMEM((n_seq * (S // _HALF), _HALF), jnp.int32),
            pltpu.VMEM((S, D), jnp.float32),
            pltpu.VMEM((S, D), jnp.float32),
            pltpu.VMEM((S, D), jnp.float32),
            pltpu.VMEM((S, D), jnp.float32),
            pltpu.VMEM((S, D), jnp.float32),
            pltpu.SemaphoreType.DMA,
            pltpu.SemaphoreType.DMA,
            pltpu.SemaphoreType.DMA,
            pltpu.SemaphoreType.DMA,
            pltpu.SemaphoreType.DMA,
            pltpu.SemaphoreType.DMA,
            pltpu.SemaphoreType.DMA,
            pltpu.SemaphoreType.DMA,
        ],
    )
    def emb_kernel(x_hbm, tok_hbm, pos_hbm, out_hbm, idx_v,
                   r0, r1, r2, r3, pos_v,
                   sg0, sg1, sg2, sg3, so0, so1, so2, so3):
        rows = (r0, r1, r2, r3)
        sg = (sg0, sg1, sg2, sg3)
        so = (so0, so1, so2, so3)
        wid = lax.axis_index("s") * _NUM_CORES + lax.axis_index("c")
        base = wid * n_seq
        pltpu.sync_copy(pos_hbm.at[pl.ds(0, S)], pos_v)
        pltpu.sync_copy(x_hbm.at[wid], idx_v)

        def issue_gather(local_seq, b):
            pltpu.async_copy(
                tok_hbm.at[idx_v.at[2 * local_seq]],
                rows[b].at[pl.ds(0, _HALF)], sg[b])
            pltpu.async_copy(
                tok_hbm.at[idx_v.at[2 * local_seq + 1]],
                rows[b].at[pl.ds(_HALF, _HALF)], sg[b])

        def wait_gather(b):
            pltpu.make_async_copy(tok_hbm.at[pl.ds(0, S)], rows[b], sg[b]).wait()

        def wait_out(b):
            pltpu.make_async_copy(rows[b], out_hbm.at[0], so[b]).wait()

        issue_gather(0, 0)
        issue_gather(1, 1)

        @pl.loop(0, n_seq, step=_NBUF)
        def _(j):
            for b in range(_NBUF):
                seq = j + b
                nb = (b + 2) % _NBUF
                wait_gather(b)
                if b < 2:
                    @pl.when(j > 0)
                    def _():
                        wait_out(nb)
                    issue_gather(seq + 2, nb)
                else:
                    wait_out(nb)

                    @pl.when(j < n_seq - _NBUF)
                    def _():
                        issue_gather(seq + 2, nb)

                @pl.loop(0, S)
                def _(r):
                    for c in range(0, D, _LANES):
                        slc = (pl.ds(r, 1), pl.ds(c, _LANES))
                        rows[b].at[*slc][...] = (
                            rows[b].at[*slc][...] + pos_v.at[*slc][...])

                pltpu.async_copy(rows[b], out_hbm.at[base + seq], so[b])

        wait_out(2)
        wait_out(3)

    return emb_kernel(x4, token_embedding, pos_embedding)


# speedup vs baseline: 1.5707x; 1.1478x over previous
import functools

import jax
import jax.numpy as jnp
from jax import lax
from jax.experimental import pallas as pl
from jax.experimental.pallas import tpu as pltpu
from jax.experimental.pallas import tpu_sc as plsc

_NUM_CORES = 2
_NUM_SUBCORES = 16
_NW = _NUM_CORES * _NUM_SUBCORES
_HALF = 100
_LANES = 16
_PADD = 128


def kernel(x, token_embedding, pos_embedding):
    B, S = x.shape
    V, D = token_embedding.shape
    n_seq = B // _NW
    x4 = x.reshape(_NW, n_seq * (S // _HALF), _HALF).astype(jnp.int32)
    tok128 = jnp.pad(token_embedding, ((0, 0), (0, _PADD - D)))

    mesh = plsc.VectorSubcoreMesh(core_axis_name="c", subcore_axis_name="s")

    @functools.partial(
        pl.kernel,
        out_type=jax.ShapeDtypeStruct((B, S, _PADD), jnp.float32),
        mesh=mesh,
        scratch_types=[
            pltpu.VMEM((n_seq * (S // _HALF), _HALF), jnp.int32),
            pltpu.VMEM((S, _PADD), jnp.float32),
            pltpu.VMEM((S, _PADD), jnp.float32),
            pltpu.VMEM((S, _PADD), jnp.float32),
            pltpu.VMEM((S, D), jnp.float32),
            pltpu.SemaphoreType.DMA,
            pltpu.SemaphoreType.DMA,
            pltpu.SemaphoreType.DMA,
            pltpu.SemaphoreType.DMA,
            pltpu.SemaphoreType.DMA,
            pltpu.SemaphoreType.DMA,
        ],
    )
    def emb_kernel(x_hbm, tok_hbm, pos_hbm, out_hbm, idx_v,
                   r0, r1, r2, pos_v, sg0, sg1, sg2, so0, so1, so2):
        rows = (r0, r1, r2)
        sg = (sg0, sg1, sg2)
        so = (so0, so1, so2)
        wid = lax.axis_index("s") * _NUM_CORES + lax.axis_index("c")
        base = wid * n_seq
        pltpu.sync_copy(pos_hbm.at[pl.ds(0, S)], pos_v)
        pltpu.sync_copy(x_hbm.at[wid], idx_v)

        def issue_gather(local_seq, b):
            pltpu.async_copy(
                tok_hbm.at[idx_v.at[2 * local_seq]],
                rows[b].at[pl.ds(0, _HALF)], sg[b])
            pltpu.async_copy(
                tok_hbm.at[idx_v.at[2 * local_seq + 1]],
                rows[b].at[pl.ds(_HALF, _HALF)], sg[b])

        def wait_gather(b):
            pltpu.make_async_copy(tok_hbm.at[pl.ds(0, S)], rows[b], sg[b]).wait()

        def wait_out(b):
            pltpu.make_async_copy(rows[b], out_hbm.at[0], so[b]).wait()

        def add_pos_and_store(seq, b):
            @pl.loop(0, S)
            def _(r):
                for c in range(0, D, _LANES):
                    slc = (pl.ds(r, 1), pl.ds(c, _LANES))
                    rows[b].at[*slc][...] = (
                        rows[b].at[*slc][...] + pos_v.at[*slc][...])

            pltpu.async_copy(rows[b], out_hbm.at[base + seq], so[b])

        issue_gather(0, 0)

        @pl.loop(0, n_seq - 2, step=3)
        def _(j):
            for b in range(3):
                seq = j + b
                nb = (b + 1) % 3
                wait_gather(b)
                if b == 2:
                    wait_out(nb)
                else:
                    @pl.when(j > 0)
                    def _():
                        wait_out(nb)
                issue_gather(seq + 1, nb)
                add_pos_and_store(seq, b)

        wait_gather(0)
        wait_out(1)
        issue_gather(n_seq - 1, 1)
        add_pos_and_store(n_seq - 2, 0)
        wait_gather(1)
        wait_out(2)
        add_pos_and_store(n_seq - 1, 1)
        wait_out(0)
        wait_out(1)

    out = emb_kernel(x4, tok128, pos_embedding)
    return out[:, :, :D]
